# baseline (speedup 1.0000x reference)
import functools

import jax
import jax.numpy as jnp
from jax import lax
from jax.experimental import pallas as pl
from jax.experimental.pallas import tpu as pltpu
from jax.experimental.pallas import tpu_sc as plsc

TIMESTEP = 0.011609977324263039

_ROWS = 16
_COLS = 4096
_LANES = 16
_NUM_CORES = 2
_SUB = _COLS // _LANES


def _make_sc_kernel():
    mesh = plsc.VectorSubcoreMesh(core_axis_name="c", subcore_axis_name="s")

    @functools.partial(
        pl.kernel,
        mesh=mesh,
        out_type=jax.ShapeDtypeStruct((_ROWS, 2 * _COLS), jnp.float32),
        scratch_types=[
            pltpu.VMEM((_COLS,), jnp.float32),
            pltpu.VMEM((2 * _COLS,), jnp.float32),
        ],
        compiler_params=pltpu.CompilerParams(needs_layout_passes=False),
    )
    def boundaries_kernel(dur_hbm, out_hbm, dur_v, out_v):
        wid = lax.axis_index("s") * _NUM_CORES + lax.axis_index("c")

        @pl.when(wid < _ROWS)
        def _():
            pltpu.sync_copy(dur_hbm.at[wid], dur_v)
            base = lax.iota(jnp.int32, _LANES) * _SUB

            def tot_body(j, acc):
                return acc + plsc.load_gather(dur_v, [base + j])

            tot = lax.fori_loop(
                0, _SUB, tot_body, jnp.zeros((_LANES,), jnp.float32)
            )
            excl = plsc.cumsum(tot) - tot

            hi = jnp.float32(_COLS)
            ts = jnp.float32(TIMESTEP)

            def out_body(j, carry):
                v = plsc.load_gather(dur_v, [base + j])
                ends = carry + v
                s_out = jnp.minimum(jnp.maximum(carry, 0.0), hi) * ts
                e_out = jnp.minimum(jnp.maximum(ends, 0.0), hi) * ts
                oi = (base + j) * 2
                plsc.store_scatter(out_v, [oi], s_out)
                plsc.store_scatter(out_v, [oi + 1], e_out)
                return ends

            lax.fori_loop(0, _SUB, out_body, excl)
            pltpu.sync_copy(out_v, out_hbm.at[wid])

    return boundaries_kernel


_sc_kernel = _make_sc_kernel()


def kernel(durations, mask):
    del mask
    out = _sc_kernel(durations)
    return out.reshape(_ROWS, _COLS, 2)
